# baseline (speedup 1.0000x reference)
import functools

import jax
import jax.numpy as jnp
from jax import lax
from jax.experimental import pallas as pl
from jax.experimental.pallas import tpu as pltpu
from jax.experimental.pallas import tpu_sc as plsc

VOCAB_ROWS = 1000000
EMBED = 64
PADDED = 128
HIDDEN = 256
CLASSES = 10
BATCH = 4096
SEQ = 200

NUM_CORES = 2
NUM_SUBCORES = 16
NUM_WORKERS = NUM_CORES * NUM_SUBCORES
ROWS_PER_W = BATCH // NUM_WORKERS

NBUF = 3

OUT_PAD = 128

REPACK_BLK = 10000


def _repack_body(in_ref, out_ref):
    out_ref[:, :EMBED] = in_ref[...]
    out_ref[:, EMBED:] = jnp.zeros((REPACK_BLK, PADDED - EMBED), jnp.float32)


def _repack(table):
    return pl.pallas_call(
        _repack_body,
        grid=(VOCAB_ROWS // REPACK_BLK,),
        in_specs=[pl.BlockSpec((REPACK_BLK, EMBED), lambda i: (i, 0))],
        out_specs=pl.BlockSpec((REPACK_BLK, PADDED), lambda i: (i, 0)),
        out_shape=jax.ShapeDtypeStruct((VOCAB_ROWS, PADDED), jnp.float32),
    )(table)


def _pool_body(x_hbm, table_hbm, doc_hbm, idx_v, doc_v, bufs, sems):
    wid = lax.axis_index("s") * NUM_CORES + lax.axis_index("c")
    base = wid * ROWS_PER_W
    pltpu.sync_copy(x_hbm.at[pl.ds(base, ROWS_PER_W)], idx_v)

    lanes = lax.broadcasted_iota(jnp.int32, (16,), 0)

    def start_gather(b, j):
        pltpu.async_copy(
            table_hbm.at[idx_v.at[b, pl.ds(0, 128)]],
            bufs[j].at[pl.ds(0, 128)], sems[j])
        pltpu.async_copy(
            table_hbm.at[idx_v.at[b, pl.ds(128, 72)]],
            bufs[j].at[pl.ds(128, 72)], sems[j])

    def wait_gather(j):
        pltpu.make_async_copy(
            table_hbm.at[pl.ds(0, SEQ)], bufs[j], sems[j]).wait()

    def process_row(b, j):
        rows_v = bufs[j]
        cnt = jnp.zeros((16,), jnp.float32)
        one = jnp.ones((16,), jnp.float32)
        zero16 = jnp.zeros((16,), jnp.float32)
        for c in range(12):
            v = idx_v[b, pl.ds(c * 16, 16)]
            cnt = cnt + jnp.where(v != 0, one, zero16)
        v = idx_v[b, pl.ds(184, 16)]
        vm = jnp.where(lanes >= 8, v, jnp.zeros((16,), jnp.int32))
        cnt = cnt + jnp.where(vm != 0, one, zero16)
        denom = jnp.maximum(jnp.sum(cnt), jnp.float32(1.0))
        inv = jnp.ones((16,), jnp.float32) / lax.broadcast_in_dim(
            denom, (16,), ())

        def acc_body(i, acc):
            a0, a1, a2, a3 = acc
            for k in range(8):
                s = i * 8 + k
                a0 = a0 + rows_v[s, pl.ds(0, 16)]
                a1 = a1 + rows_v[s, pl.ds(16, 16)]
                a2 = a2 + rows_v[s, pl.ds(32, 16)]
                a3 = a3 + rows_v[s, pl.ds(48, 16)]
            return (a0, a1, a2, a3)

        zero = jnp.zeros((16,), jnp.float32)
        a0, a1, a2, a3 = lax.fori_loop(
            0, SEQ // 8, acc_body, (zero, zero, zero, zero))

        doc_v[b, pl.ds(0, 16)] = a0 * inv
        doc_v[b, pl.ds(16, 16)] = a1 * inv
        doc_v[b, pl.ds(32, 16)] = a2 * inv
        doc_v[b, pl.ds(48, 16)] = a3 * inv

    for j in range(NBUF):
        start_gather(j, j)

    def group_body(g, carry):
        for j in range(NBUF):
            b = g * NBUF + j
            wait_gather(j)
            process_row(b, j)
            start_gather(b + NBUF, j)
        return carry

    n_steady = ROWS_PER_W // NBUF - 1
    lax.fori_loop(0, n_steady, group_body, 0)

    done = n_steady * NBUF
    for b in range(done, ROWS_PER_W):
        j = b % NBUF
        wait_gather(j)
        process_row(b, j)
        nxt = b + NBUF
        if done + NBUF <= nxt < ROWS_PER_W:
            start_gather(nxt, j)

    pltpu.sync_copy(doc_v, doc_hbm.at[pl.ds(base, ROWS_PER_W)])


@functools.partial(
    pl.kernel,
    out_type=jax.ShapeDtypeStruct((BATCH, EMBED), jnp.float32),
    mesh=plsc.VectorSubcoreMesh(core_axis_name="c", subcore_axis_name="s"),
    scratch_types=[
        pltpu.VMEM((ROWS_PER_W, SEQ), jnp.int32),
        pltpu.VMEM((ROWS_PER_W, EMBED), jnp.float32),
        [pltpu.VMEM((SEQ, PADDED), jnp.float32) for _ in range(NBUF)],
        [pltpu.SemaphoreType.DMA for _ in range(NBUF)],
    ],
    compiler_params=pltpu.CompilerParams(needs_layout_passes=False),
)
def _pool(x_hbm, table_hbm, doc_hbm, idx_v, doc_v, bufs, sems):
    _pool_body(x_hbm, table_hbm, doc_hbm, idx_v, doc_v, bufs, sems)


def _mlp_body(doc_ref, w1_ref, b1_ref, w2_ref, b2_ref, out_ref):
    doc = doc_ref[...]
    h = lax.dot_general(doc, w1_ref[...], (((1,), (1,)), ((), ())),
                        preferred_element_type=jnp.float32)
    h = jnp.maximum(h + b1_ref[...], 0.0)
    out = lax.dot_general(h, w2_ref[...], (((1,), (1,)), ((), ())),
                          preferred_element_type=jnp.float32)
    out_ref[...] = out + b2_ref[...]


def _mlp(doc, W1, b1, W2p, b2p):
    blk = 512
    grid = BATCH // blk
    return pl.pallas_call(
        _mlp_body,
        grid=(grid,),
        in_specs=[
            pl.BlockSpec((blk, EMBED), lambda i: (i, 0)),
            pl.BlockSpec((HIDDEN, EMBED), lambda i: (0, 0)),
            pl.BlockSpec((1, HIDDEN), lambda i: (0, 0)),
            pl.BlockSpec((OUT_PAD, HIDDEN), lambda i: (0, 0)),
            pl.BlockSpec((1, OUT_PAD), lambda i: (0, 0)),
        ],
        out_specs=pl.BlockSpec((blk, OUT_PAD), lambda i: (i, 0)),
        out_shape=jax.ShapeDtypeStruct((BATCH, OUT_PAD), jnp.float32),
    )(doc, W1, b1, W2p, b2p)


@jax.jit
def kernel(x, emb_table, W1, b1, W2, b2):
    x = x.astype(jnp.int32)
    table_pad = _repack(emb_table)
    doc = _pool(x, table_pad)
    W2p = jnp.zeros((OUT_PAD, HIDDEN), jnp.float32).at[:CLASSES].set(W2)
    b2p = jnp.zeros((OUT_PAD,), jnp.float32).at[:CLASSES].set(b2)
    out = _mlp(doc, W1, b1.reshape(1, HIDDEN), W2p, b2p.reshape(1, OUT_PAD))
    return out[:, :CLASSES]


# speedup vs baseline: 1.8563x; 1.8563x over previous
import functools

import jax
import jax.numpy as jnp
from jax import lax
from jax.experimental import pallas as pl
from jax.experimental.pallas import tpu as pltpu
from jax.experimental.pallas import tpu_sc as plsc

VOCAB_ROWS = 1000000
EMBED = 64
PADDED = 128
HIDDEN = 256
CLASSES = 10
BATCH = 4096
SEQ = 200

NUM_CORES = 2
NUM_SUBCORES = 16
NUM_WORKERS = NUM_CORES * NUM_SUBCORES
ROWS_PER_W = BATCH // NUM_WORKERS

NBUF = 3

OUT_PAD = 128

REPACK_BLK = 8192


def _repack_body(in_ref, out_ref):
    out_ref[:, :EMBED] = in_ref[...].T
    out_ref[:, EMBED:] = jnp.zeros((REPACK_BLK, PADDED - EMBED), jnp.float32)


def _repack(table_t):
    grid = (VOCAB_ROWS + REPACK_BLK - 1) // REPACK_BLK
    return pl.pallas_call(
        _repack_body,
        grid=(grid,),
        in_specs=[pl.BlockSpec((EMBED, REPACK_BLK), lambda i: (0, i))],
        out_specs=pl.BlockSpec((REPACK_BLK, PADDED), lambda i: (i, 0)),
        out_shape=jax.ShapeDtypeStruct((VOCAB_ROWS, PADDED), jnp.float32),
    )(table_t)


def _pool_body(x_hbm, table_hbm, doc_hbm, idx_v, doc_v, bufs, sems):
    wid = lax.axis_index("s") * NUM_CORES + lax.axis_index("c")
    base = wid * ROWS_PER_W
    pltpu.sync_copy(x_hbm.at[pl.ds(base, ROWS_PER_W)], idx_v)

    lanes = lax.broadcasted_iota(jnp.int32, (16,), 0)

    def start_gather(b, j):
        pltpu.async_copy(
            table_hbm.at[idx_v.at[b, pl.ds(0, 128)]],
            bufs[j].at[pl.ds(0, 128)], sems[j])
        pltpu.async_copy(
            table_hbm.at[idx_v.at[b, pl.ds(128, 72)]],
            bufs[j].at[pl.ds(128, 72)], sems[j])

    def wait_gather(j):
        pltpu.make_async_copy(
            table_hbm.at[pl.ds(0, SEQ)], bufs[j], sems[j]).wait()

    def process_row(b, j):
        rows_v = bufs[j]
        cnt = jnp.zeros((16,), jnp.float32)
        one = jnp.ones((16,), jnp.float32)
        zero16 = jnp.zeros((16,), jnp.float32)
        for c in range(12):
            v = idx_v[b, pl.ds(c * 16, 16)]
            cnt = cnt + jnp.where(v != 0, one, zero16)
        v = idx_v[b, pl.ds(184, 16)]
        vm = jnp.where(lanes >= 8, v, jnp.zeros((16,), jnp.int32))
        cnt = cnt + jnp.where(vm != 0, one, zero16)
        denom = jnp.maximum(jnp.sum(cnt), jnp.float32(1.0))
        inv = jnp.ones((16,), jnp.float32) / lax.broadcast_in_dim(
            denom, (16,), ())

        def acc_body(i, acc):
            a0, a1, a2, a3 = acc
            for k in range(8):
                s = i * 8 + k
                a0 = a0 + rows_v[s, pl.ds(0, 16)]
                a1 = a1 + rows_v[s, pl.ds(16, 16)]
                a2 = a2 + rows_v[s, pl.ds(32, 16)]
                a3 = a3 + rows_v[s, pl.ds(48, 16)]
            return (a0, a1, a2, a3)

        zero = jnp.zeros((16,), jnp.float32)
        a0, a1, a2, a3 = lax.fori_loop(
            0, SEQ // 8, acc_body, (zero, zero, zero, zero))

        doc_v[b, pl.ds(0, 16)] = a0 * inv
        doc_v[b, pl.ds(16, 16)] = a1 * inv
        doc_v[b, pl.ds(32, 16)] = a2 * inv
        doc_v[b, pl.ds(48, 16)] = a3 * inv

    for j in range(NBUF):
        start_gather(j, j)

    def group_body(g, carry):
        for j in range(NBUF):
            b = g * NBUF + j
            wait_gather(j)
            process_row(b, j)
            start_gather(b + NBUF, j)
        return carry

    n_steady = ROWS_PER_W // NBUF - 1
    lax.fori_loop(0, n_steady, group_body, 0)

    done = n_steady * NBUF
    for b in range(done, ROWS_PER_W):
        j = b % NBUF
        wait_gather(j)
        process_row(b, j)
        nxt = b + NBUF
        if done + NBUF <= nxt < ROWS_PER_W:
            start_gather(nxt, j)

    pltpu.sync_copy(doc_v, doc_hbm.at[pl.ds(base, ROWS_PER_W)])


@functools.partial(
    pl.kernel,
    out_type=jax.ShapeDtypeStruct((BATCH, EMBED), jnp.float32),
    mesh=plsc.VectorSubcoreMesh(core_axis_name="c", subcore_axis_name="s"),
    scratch_types=[
        pltpu.VMEM((ROWS_PER_W, SEQ), jnp.int32),
        pltpu.VMEM((ROWS_PER_W, EMBED), jnp.float32),
        [pltpu.VMEM((SEQ, PADDED), jnp.float32) for _ in range(NBUF)],
        [pltpu.SemaphoreType.DMA for _ in range(NBUF)],
    ],
    compiler_params=pltpu.CompilerParams(needs_layout_passes=False),
)
def _pool(x_hbm, table_hbm, doc_hbm, idx_v, doc_v, bufs, sems):
    _pool_body(x_hbm, table_hbm, doc_hbm, idx_v, doc_v, bufs, sems)


def _mlp_body(doc_ref, w1_ref, b1_ref, w2_ref, b2_ref, out_ref):
    doc = doc_ref[...]
    h = lax.dot_general(doc, w1_ref[...], (((1,), (1,)), ((), ())),
                        preferred_element_type=jnp.float32)
    h = jnp.maximum(h + b1_ref[...], 0.0)
    out = lax.dot_general(h, w2_ref[...], (((1,), (1,)), ((), ())),
                          preferred_element_type=jnp.float32)
    out_ref[...] = out + b2_ref[...]


def _mlp(doc, W1, b1, W2p, b2p):
    blk = 512
    grid = BATCH // blk
    return pl.pallas_call(
        _mlp_body,
        grid=(grid,),
        in_specs=[
            pl.BlockSpec((blk, EMBED), lambda i: (i, 0)),
            pl.BlockSpec((HIDDEN, EMBED), lambda i: (0, 0)),
            pl.BlockSpec((1, HIDDEN), lambda i: (0, 0)),
            pl.BlockSpec((OUT_PAD, HIDDEN), lambda i: (0, 0)),
            pl.BlockSpec((1, OUT_PAD), lambda i: (0, 0)),
        ],
        out_specs=pl.BlockSpec((blk, OUT_PAD), lambda i: (i, 0)),
        out_shape=jax.ShapeDtypeStruct((BATCH, OUT_PAD), jnp.float32),
    )(doc, W1, b1, W2p, b2p)


@jax.jit
def kernel(x, emb_table, W1, b1, W2, b2):
    x = x.astype(jnp.int32)
    table_pad = _repack(emb_table.T)
    doc = _pool(x, table_pad)
    W2p = jnp.zeros((OUT_PAD, HIDDEN), jnp.float32).at[:CLASSES].set(W2)
    b2p = jnp.zeros((OUT_PAD,), jnp.float32).at[:CLASSES].set(b2)
    out = _mlp(doc, W1, b1.reshape(1, HIDDEN), W2p, b2p.reshape(1, OUT_PAD))
    return out[:, :CLASSES]
